# Initial kernel scaffold; baseline (speedup 1.0000x reference)
import jax
import jax.numpy as jnp
from jax.experimental import pallas as pl


def kernel(coor_t, coor_h, coor_w, grid0, grid1, grid2, W_lin, b_lin):
    raise NotImplementedError("write your pallas kernel here")


# speedup vs baseline: 4.1195x; 4.1195x over previous
import functools

import jax
import jax.numpy as jnp
from jax import lax
from jax.experimental import pallas as pl
from jax.experimental.pallas import tpu as pltpu
from jax.experimental.pallas import tpu_sc as plsc

_LEVELS = [(120, 9, 16, 64), (60, 9, 16, 128), (30, 9, 16, 256)]
_OFFS = [0, 64, 192]
_COUT = 128
_KP = 4
_D = _KP * _COUT

_NW = 32
_PT = 512
_B = 2
_NB = _PT // _B


def _proj_body(g_ref, w_ref, b_ref, o_ref):
    o_ref[...] = (
        jnp.dot(g_ref[...], w_ref[...], preferred_element_type=jnp.float32)
        + b_ref[...]
    )


def _project(g4, w_t, b3):
    r4, cg = g4.shape
    br = 640
    return pl.pallas_call(
        _proj_body,
        grid=(r4 // br,),
        in_specs=[
            pl.BlockSpec((br, cg), lambda i: (i, 0)),
            pl.BlockSpec((cg, _COUT), lambda i: (0, 0)),
            pl.BlockSpec((1, _COUT), lambda i: (0, 0)),
        ],
        out_specs=pl.BlockSpec((br, _COUT), lambda i: (i, 0)),
        out_shape=jax.ShapeDtypeStruct((r4, _COUT), jnp.float32),
    )(g4, w_t, b3)


def _sc_body(pg0, pg1, pg2, idx0, idx1, idx2, wgt, out,
             idxv0, idxv1, idxv2, wgv,
             b0a, b0b, b1a, b1b, b2a, b2b, oba, obb,
             g0a, g0b, g1a, g1b, g2a, g2b, soa, sob):
    cid = lax.axis_index("c")
    sid = lax.axis_index("s")
    wid = sid * 2 + cid
    base = wid * _PT

    pltpu.sync_copy(idx0.at[wid], idxv0)
    pltpu.sync_copy(idx1.at[wid], idxv1)
    pltpu.sync_copy(idx2.at[wid], idxv2)
    pltpu.sync_copy(wgt.at[wid], wgv)

    def fire(g, bufs, sems):
        sl = pl.ds(g * (_B * 8), _B * 8)
        pltpu.make_async_copy(pg0.at[idxv0.at[sl]], bufs[0], sems[0]).start()
        pltpu.make_async_copy(pg1.at[idxv1.at[sl]], bufs[1], sems[1]).start()
        pltpu.make_async_copy(pg2.at[idxv2.at[sl]], bufs[2], sems[2]).start()

    def drain(g, bufs, sems):
        sl = pl.ds(g * (_B * 8), _B * 8)
        pltpu.make_async_copy(pg0.at[idxv0.at[sl]], bufs[0], sems[0]).wait()
        pltpu.make_async_copy(pg1.at[idxv1.at[sl]], bufs[1], sems[1]).wait()
        pltpu.make_async_copy(pg2.at[idxv2.at[sl]], bufs[2], sems[2]).wait()

    def out_slice(g):
        return out.at[pl.ds(base + g * _B, _B)]

    def compute(g, bufs, ob):
        nwv = 24 * _B // 16
        wvecs = [wgv[pl.ds(g * (24 * _B) + k * 16, 16)] for k in range(nwv)]
        for p in range(_B):
            ws = []
            for m in range(24):
                q = p * 24 + m
                ws.append(jnp.full((16,), wvecs[q // 16][q % 16], jnp.float32))

            def chunk(c, _, p=p, ws=ws):
                off = c * 16
                acc = ws[0] * bufs[0][p * 8, pl.ds(off, 16)]
                for j in range(1, 8):
                    acc = acc + ws[j] * bufs[0][p * 8 + j, pl.ds(off, 16)]
                for j in range(8):
                    acc = acc + ws[8 + j] * bufs[1][p * 8 + j, pl.ds(off, 16)]
                for j in range(8):
                    acc = acc + ws[16 + j] * bufs[2][p * 8 + j, pl.ds(off, 16)]
                ob[p, pl.ds(off, 16)] = acc
                return _

            lax.fori_loop(0, _D // 16, chunk, 0)

    bufs_a = (b0a, b1a, b2a)
    bufs_b = (b0b, b1b, b2b)
    sems_a = (g0a, g1a, g2a)
    sems_b = (g0b, g1b, g2b)

    fire(0, bufs_a, sems_a)

    def body(gg, carry):
        g0 = gg * 2
        g1 = g0 + 1
        fire(g1, bufs_b, sems_b)

        drain(g0, bufs_a, sems_a)

        @pl.when(gg > 0)
        def _():
            pltpu.make_async_copy(oba, out_slice(g0), soa).wait()

        compute(g0, bufs_a, oba)
        pltpu.make_async_copy(oba, out_slice(g0), soa).start()

        @pl.when(g0 + 2 < _NB)
        def _():
            fire(g0 + 2, bufs_a, sems_a)

        drain(g1, bufs_b, sems_b)

        @pl.when(gg > 0)
        def _():
            pltpu.make_async_copy(obb, out_slice(g1), sob).wait()

        compute(g1, bufs_b, obb)
        pltpu.make_async_copy(obb, out_slice(g1), sob).start()
        return carry

    lax.fori_loop(0, _NB // 2, body, 0)

    pltpu.make_async_copy(oba, out_slice(_NB - 2), soa).wait()
    pltpu.make_async_copy(obb, out_slice(_NB - 1), sob).wait()


def _sc_gather(p_total, pgs, idxs, wgts):
    f32 = jnp.float32
    k = functools.partial(
        pl.kernel,
        out_type=jax.ShapeDtypeStruct((p_total, _D), f32),
        mesh=plsc.VectorSubcoreMesh(core_axis_name="c", subcore_axis_name="s"),
        scratch_types=[
            pltpu.VMEM((_NB * _B * 8,), jnp.int32),
            pltpu.VMEM((_NB * _B * 8,), jnp.int32),
            pltpu.VMEM((_NB * _B * 8,), jnp.int32),
            pltpu.VMEM((_PT * 24,), f32),
            pltpu.VMEM((_B * 8, _D), f32),
            pltpu.VMEM((_B * 8, _D), f32),
            pltpu.VMEM((_B * 8, _D), f32),
            pltpu.VMEM((_B * 8, _D), f32),
            pltpu.VMEM((_B * 8, _D), f32),
            pltpu.VMEM((_B * 8, _D), f32),
            pltpu.VMEM((_B, _D), f32),
            pltpu.VMEM((_B, _D), f32),
            pltpu.SemaphoreType.DMA,
            pltpu.SemaphoreType.DMA,
            pltpu.SemaphoreType.DMA,
            pltpu.SemaphoreType.DMA,
            pltpu.SemaphoreType.DMA,
            pltpu.SemaphoreType.DMA,
            pltpu.SemaphoreType.DMA,
            pltpu.SemaphoreType.DMA,
        ],
    )(_sc_body)
    return k(pgs[0], pgs[1], pgs[2], idxs[0], idxs[1], idxs[2], wgts)


def _axis_iw(c, size):
    p = (c + 1.0) * 0.5 * (size - 1)
    f = jnp.floor(p)
    w1 = p - f
    i0 = jnp.clip(f.astype(jnp.int32), 0, size - 1)
    i1 = jnp.clip(f.astype(jnp.int32) + 1, 0, size - 1)
    return i0, i1, w1


def kernel(coor_t, coor_h, coor_w, grid0, grid1, grid2, W_lin, b_lin):
    n, t = coor_t.shape
    h = coor_h.shape[1]
    w = coor_w.shape[1]
    p_total = n * t * h * w

    b3 = (b_lin.astype(jnp.float32) / 3.0).reshape(1, _COUT)

    pgs, idxs, wgts = [], [], []
    for gp, off, (tg, hg, wg, cg) in zip((grid0, grid1, grid2), _OFFS, _LEVELS):
        rows = tg * hg * wg
        g4 = gp.reshape(rows * _KP, cg)
        w_t = W_lin[:, off:off + cg].T.astype(jnp.float32)
        pg = _project(g4, w_t, b3).reshape(rows, _D)
        pgs.append(pg)

        z0, z1, wz = _axis_iw(coor_t, tg)
        y0, y1, wy = _axis_iw(coor_h, hg)
        x0, x1, wx = _axis_iw(coor_w, wg)
        zi = jnp.stack([z0, z1], -1)
        yi = jnp.stack([y0, y1], -1)
        xi = jnp.stack([x0, x1], -1)
        wzv = jnp.stack([1.0 - wz, wz], -1)
        wyv = jnp.stack([1.0 - wy, wy], -1)
        wxv = jnp.stack([1.0 - wx, wx], -1)
        idx = (zi[:, :, None, None, :, None, None] * (hg * wg)
               + yi[:, None, :, None, None, :, None] * wg
               + xi[:, None, None, :, None, None, :])
        wgt = (wzv[:, :, None, None, :, None, None]
               * wyv[:, None, :, None, None, :, None]
               * wxv[:, None, None, :, None, None, :])
        idxs.append(idx.reshape(_NW, _NB * _B * 8).astype(jnp.int32))
        wgts.append(wgt.reshape(_NW, _PT, 8).astype(jnp.float32))

    wgt_all = jnp.concatenate(wgts, axis=-1).reshape(_NW, _PT * 24)
    out = _sc_gather(p_total, pgs, idxs, wgt_all)
    return out.reshape(n, t, h, w, _KP * _COUT)


# speedup vs baseline: 4.1255x; 1.0015x over previous
import functools

import jax
import jax.numpy as jnp
from jax import lax
from jax.experimental import pallas as pl
from jax.experimental.pallas import tpu as pltpu
from jax.experimental.pallas import tpu_sc as plsc

_LEVELS = [(120, 9, 16, 64), (60, 9, 16, 128), (30, 9, 16, 256)]
_OFFS = [0, 64, 192]
_COUT = 128
_KP = 4
_D = _KP * _COUT

_NW = 32
_PT = 512
_B = 2
_NB = _PT // _B


def _proj_body(g_ref, w_ref, b_ref, o_ref):
    o_ref[...] = (
        jnp.dot(g_ref[...], w_ref[...], preferred_element_type=jnp.float32)
        + b_ref[...]
    )


def _project(g4, w_t, b3):
    r4, cg = g4.shape
    br = 640
    return pl.pallas_call(
        _proj_body,
        grid=(r4 // br,),
        in_specs=[
            pl.BlockSpec((br, cg), lambda i: (i, 0)),
            pl.BlockSpec((cg, _COUT), lambda i: (0, 0)),
            pl.BlockSpec((1, _COUT), lambda i: (0, 0)),
        ],
        out_specs=pl.BlockSpec((br, _COUT), lambda i: (i, 0)),
        out_shape=jax.ShapeDtypeStruct((r4, _COUT), jnp.float32),
    )(g4, w_t, b3)


def _sc_body(pg0, pg1, pg2, idx0, idx1, idx2, wgt, out,
             idxv0, idxv1, idxv2, wgv,
             b0a, b0b, b1a, b1b, b2a, b2b, oba, obb,
             g0a, g0b, g1a, g1b, g2a, g2b, soa, sob):
    cid = lax.axis_index("c")
    sid = lax.axis_index("s")
    wid = sid * 2 + cid
    base = wid * _PT

    pltpu.sync_copy(idx0.at[wid], idxv0)
    pltpu.sync_copy(idx1.at[wid], idxv1)
    pltpu.sync_copy(idx2.at[wid], idxv2)
    pltpu.sync_copy(wgt.at[wid], wgv)

    def fire(g, bufs, sems):
        sl = pl.ds(g * (_B * 8), _B * 8)
        pltpu.make_async_copy(pg0.at[idxv0.at[sl]], bufs[0], sems[0]).start()
        pltpu.make_async_copy(pg1.at[idxv1.at[sl]], bufs[1], sems[1]).start()
        pltpu.make_async_copy(pg2.at[idxv2.at[sl]], bufs[2], sems[2]).start()

    def drain(g, bufs, sems):
        sl = pl.ds(g * (_B * 8), _B * 8)
        pltpu.make_async_copy(pg0.at[idxv0.at[sl]], bufs[0], sems[0]).wait()
        pltpu.make_async_copy(pg1.at[idxv1.at[sl]], bufs[1], sems[1]).wait()
        pltpu.make_async_copy(pg2.at[idxv2.at[sl]], bufs[2], sems[2]).wait()

    def out_slice(g):
        return out.at[pl.ds(base + g * _B, _B)]

    def compute(g, bufs, ob):
        nwv = 24 * _B // 16
        wvecs = [wgv[pl.ds(g * (24 * _B) + k * 16, 16)] for k in range(nwv)]
        for p in range(_B):
            ws = []
            for m in range(24):
                q = p * 24 + m
                ws.append(jnp.full((16,), wvecs[q // 16][q % 16], jnp.float32))

            def chunk(c, _, p=p, ws=ws):
                off = c * 16
                acc = None
                for l in range(3):
                    for j in range(8):
                        v = bufs[l][p * 8 + j, pl.ds(off, 16)]
                        wv = ws[l * 8 + j]
                        acc = wv * v if acc is None else acc + wv * v
                ob[p, pl.ds(off, 16)] = acc
                return _

            lax.fori_loop(0, _D // 16, chunk, 0, unroll=2)

    bufs_a = (b0a, b1a, b2a)
    bufs_b = (b0b, b1b, b2b)
    sems_a = (g0a, g1a, g2a)
    sems_b = (g0b, g1b, g2b)

    fire(0, bufs_a, sems_a)

    def body(gg, carry):
        g0 = gg * 2
        g1 = g0 + 1
        fire(g1, bufs_b, sems_b)

        drain(g0, bufs_a, sems_a)

        @pl.when(gg > 0)
        def _():
            pltpu.make_async_copy(oba, out_slice(g0), soa).wait()

        compute(g0, bufs_a, oba)
        pltpu.make_async_copy(oba, out_slice(g0), soa).start()

        @pl.when(g0 + 2 < _NB)
        def _():
            fire(g0 + 2, bufs_a, sems_a)

        drain(g1, bufs_b, sems_b)

        @pl.when(gg > 0)
        def _():
            pltpu.make_async_copy(obb, out_slice(g1), sob).wait()

        compute(g1, bufs_b, obb)
        pltpu.make_async_copy(obb, out_slice(g1), sob).start()
        return carry

    lax.fori_loop(0, _NB // 2, body, 0)

    pltpu.make_async_copy(oba, out_slice(_NB - 2), soa).wait()
    pltpu.make_async_copy(obb, out_slice(_NB - 1), sob).wait()


def _sc_gather(p_total, pgs, idxs, wgts):
    f32 = jnp.float32
    k = functools.partial(
        pl.kernel,
        out_type=jax.ShapeDtypeStruct((p_total, _D), f32),
        mesh=plsc.VectorSubcoreMesh(core_axis_name="c", subcore_axis_name="s"),
        scratch_types=[
            pltpu.VMEM((_NB * _B * 8,), jnp.int32),
            pltpu.VMEM((_NB * _B * 8,), jnp.int32),
            pltpu.VMEM((_NB * _B * 8,), jnp.int32),
            pltpu.VMEM((_PT * 24,), f32),
            pltpu.VMEM((_B * 8, _D), f32),
            pltpu.VMEM((_B * 8, _D), f32),
            pltpu.VMEM((_B * 8, _D), f32),
            pltpu.VMEM((_B * 8, _D), f32),
            pltpu.VMEM((_B * 8, _D), f32),
            pltpu.VMEM((_B * 8, _D), f32),
            pltpu.VMEM((_B, _D), f32),
            pltpu.VMEM((_B, _D), f32),
            pltpu.SemaphoreType.DMA,
            pltpu.SemaphoreType.DMA,
            pltpu.SemaphoreType.DMA,
            pltpu.SemaphoreType.DMA,
            pltpu.SemaphoreType.DMA,
            pltpu.SemaphoreType.DMA,
            pltpu.SemaphoreType.DMA,
            pltpu.SemaphoreType.DMA,
        ],
    )(_sc_body)
    return k(pgs[0], pgs[1], pgs[2], idxs[0], idxs[1], idxs[2], wgts)


def _axis_iw(c, size):
    p = (c + 1.0) * 0.5 * (size - 1)
    f = jnp.floor(p)
    w1 = p - f
    i0 = jnp.clip(f.astype(jnp.int32), 0, size - 1)
    i1 = jnp.clip(f.astype(jnp.int32) + 1, 0, size - 1)
    return i0, i1, w1


def kernel(coor_t, coor_h, coor_w, grid0, grid1, grid2, W_lin, b_lin):
    n, t = coor_t.shape
    h = coor_h.shape[1]
    w = coor_w.shape[1]
    p_total = n * t * h * w

    w_p = W_lin
    b3 = (b_lin.astype(jnp.float32) / 3.0).reshape(1, _COUT)

    pgs, idxs, wgts = [], [], []
    for gp, off, (tg, hg, wg, cg) in zip((grid0, grid1, grid2), _OFFS, _LEVELS):
        rows = tg * hg * wg
        g4 = gp.reshape(rows * _KP, cg)
        w_t = w_p[:, off:off + cg].T.astype(jnp.float32)
        pg = _project(g4, w_t, b3).reshape(rows, _D)
        pgs.append(pg)

        z0, z1, wz = _axis_iw(coor_t, tg)
        y0, y1, wy = _axis_iw(coor_h, hg)
        x0, x1, wx = _axis_iw(coor_w, wg)
        zi = jnp.stack([z0, z1], -1)
        yi = jnp.stack([y0, y1], -1)
        xi = jnp.stack([x0, x1], -1)
        wzv = jnp.stack([1.0 - wz, wz], -1)
        wyv = jnp.stack([1.0 - wy, wy], -1)
        wxv = jnp.stack([1.0 - wx, wx], -1)
        idx = (zi[:, :, None, None, :, None, None] * (hg * wg)
               + yi[:, None, :, None, None, :, None] * wg
               + xi[:, None, None, :, None, None, :])
        wgt = (wzv[:, :, None, None, :, None, None]
               * wyv[:, None, :, None, None, :, None]
               * wxv[:, None, None, :, None, None, :])
        idxs.append(idx.reshape(_NW, _NB * _B * 8).astype(jnp.int32))
        wgts.append(wgt.reshape(_NW, _PT, 8).astype(jnp.float32))

    wgt_all = jnp.concatenate(wgts, axis=-1).reshape(_NW, _PT * 24)
    out = _sc_gather(p_total, pgs, idxs, wgt_all)
    return out.reshape(n, t, h, w, _KP * _COUT)


# speedup vs baseline: 4.7872x; 1.1604x over previous
import functools

import jax
import jax.numpy as jnp
from jax import lax
from jax.experimental import pallas as pl
from jax.experimental.pallas import tpu as pltpu
from jax.experimental.pallas import tpu_sc as plsc

_LEVELS = [(120, 9, 16, 64), (60, 9, 16, 128), (30, 9, 16, 256)]
_OFFS = [0, 64, 192]
_COUT = 128
_KP = 4
_D = _KP * _COUT

_NW = 32
_PT = 512
_B = 2
_NB = _PT // _B
_R = 24


def _proj_body(g_ref, w_ref, b_ref, o_ref):
    o_ref[...] = (
        jnp.dot(g_ref[...], w_ref[...], preferred_element_type=jnp.float32)
        + b_ref[...]
    )


def _project(g4, w_t, b3):
    r4, cg = g4.shape
    br = 640
    return pl.pallas_call(
        _proj_body,
        grid=(r4 // br,),
        in_specs=[
            pl.BlockSpec((br, cg), lambda i: (i, 0)),
            pl.BlockSpec((cg, _COUT), lambda i: (0, 0)),
            pl.BlockSpec((1, _COUT), lambda i: (0, 0)),
        ],
        out_specs=pl.BlockSpec((br, _COUT), lambda i: (i, 0)),
        out_shape=jax.ShapeDtypeStruct((r4, _COUT), jnp.float32),
    )(g4, w_t, b3)


def _sc_body(pg, idx, wgt, out,
             idxv, wgv, bfa, bfb, oba, obb, sga, sgb, soa, sob):
    cid = lax.axis_index("c")
    sid = lax.axis_index("s")
    wid = sid * 2 + cid
    base = wid * _PT

    pltpu.sync_copy(idx.at[wid], idxv)
    pltpu.sync_copy(wgt.at[wid], wgv)

    def fire(g, buf, sem):
        sl = pl.ds(g * (_B * _R), _B * _R)
        pltpu.make_async_copy(pg.at[idxv.at[sl]], buf, sem).start()

    def drain(g, buf, sem):
        sl = pl.ds(g * (_B * _R), _B * _R)
        pltpu.make_async_copy(pg.at[idxv.at[sl]], buf, sem).wait()

    def out_slice(g):
        return out.at[pl.ds(base + g * _B, _B)]

    def compute(g, buf, ob):
        nwv = _R * _B // 16
        wvecs = [wgv[pl.ds(g * (_R * _B) + k * 16, 16)] for k in range(nwv)]
        for p in range(_B):
            ws = []
            for m in range(_R):
                q = p * _R + m
                ws.append(jnp.full((16,), wvecs[q // 16][q % 16], jnp.float32))

            def chunk(c, _, p=p, ws=ws):
                off = c * 16
                acc = None
                for m in range(_R):
                    v = buf[p * _R + m, pl.ds(off, 16)]
                    acc = ws[m] * v if acc is None else acc + ws[m] * v
                ob[p, pl.ds(off, 16)] = acc
                return _

            lax.fori_loop(0, _D // 16, chunk, 0, unroll=2)

    fire(0, bfa, sga)

    def body(gg, carry):
        g0 = gg * 2
        g1 = g0 + 1
        fire(g1, bfb, sgb)

        drain(g0, bfa, sga)

        @pl.when(gg > 0)
        def _():
            pltpu.make_async_copy(oba, out_slice(g0), soa).wait()

        compute(g0, bfa, oba)
        pltpu.make_async_copy(oba, out_slice(g0), soa).start()

        @pl.when(g0 + 2 < _NB)
        def _():
            fire(g0 + 2, bfa, sga)

        drain(g1, bfb, sgb)

        @pl.when(gg > 0)
        def _():
            pltpu.make_async_copy(obb, out_slice(g1), sob).wait()

        compute(g1, bfb, obb)
        pltpu.make_async_copy(obb, out_slice(g1), sob).start()
        return carry

    lax.fori_loop(0, _NB // 2, body, 0)

    pltpu.make_async_copy(oba, out_slice(_NB - 2), soa).wait()
    pltpu.make_async_copy(obb, out_slice(_NB - 1), sob).wait()


def _sc_gather(p_total, pg_all, idx_all, wgt_all):
    f32 = jnp.float32
    k = functools.partial(
        pl.kernel,
        out_type=jax.ShapeDtypeStruct((p_total, _D), f32),
        mesh=plsc.VectorSubcoreMesh(core_axis_name="c", subcore_axis_name="s"),
        scratch_types=[
            pltpu.VMEM((_NB * _B * _R,), jnp.int32),
            pltpu.VMEM((_PT * _R,), f32),
            pltpu.VMEM((_B * _R, _D), f32),
            pltpu.VMEM((_B * _R, _D), f32),
            pltpu.VMEM((_B, _D), f32),
            pltpu.VMEM((_B, _D), f32),
            pltpu.SemaphoreType.DMA,
            pltpu.SemaphoreType.DMA,
            pltpu.SemaphoreType.DMA,
            pltpu.SemaphoreType.DMA,
        ],
    )(_sc_body)
    return k(pg_all, idx_all, wgt_all)


def _axis_iw(c, size):
    p = (c + 1.0) * 0.5 * (size - 1)
    f = jnp.floor(p)
    w1 = p - f
    i0 = jnp.clip(f.astype(jnp.int32), 0, size - 1)
    i1 = jnp.clip(f.astype(jnp.int32) + 1, 0, size - 1)
    return i0, i1, w1


def kernel(coor_t, coor_h, coor_w, grid0, grid1, grid2, W_lin, b_lin):
    n, t = coor_t.shape
    h = coor_h.shape[1]
    w = coor_w.shape[1]
    p_total = n * t * h * w

    w_p = W_lin
    b3 = (b_lin.astype(jnp.float32) / 3.0).reshape(1, _COUT)

    pgs, idxs, wgts = [], [], []
    row_base = 0
    for gp, off, (tg, hg, wg, cg) in zip((grid0, grid1, grid2), _OFFS, _LEVELS):
        rows = tg * hg * wg
        g4 = gp.reshape(rows * _KP, cg)
        w_t = w_p[:, off:off + cg].T.astype(jnp.float32)
        pg = _project(g4, w_t, b3).reshape(rows, _D)
        pgs.append(pg)

        z0, z1, wz = _axis_iw(coor_t, tg)
        y0, y1, wy = _axis_iw(coor_h, hg)
        x0, x1, wx = _axis_iw(coor_w, wg)
        zi = jnp.stack([z0, z1], -1)
        yi = jnp.stack([y0, y1], -1)
        xi = jnp.stack([x0, x1], -1)
        wzv = jnp.stack([1.0 - wz, wz], -1)
        wyv = jnp.stack([1.0 - wy, wy], -1)
        wxv = jnp.stack([1.0 - wx, wx], -1)
        idx = (zi[:, :, None, None, :, None, None] * (hg * wg)
               + yi[:, None, :, None, None, :, None] * wg
               + xi[:, None, None, :, None, None, :])
        wgt = (wzv[:, :, None, None, :, None, None]
               * wyv[:, None, :, None, None, :, None]
               * wxv[:, None, None, :, None, None, :])
        idxs.append(idx.reshape(n, t, h, w, 8).astype(jnp.int32) + row_base)
        wgts.append(wgt.reshape(_NW, _PT, 8).astype(jnp.float32))
        row_base += rows

    pg_all = jnp.concatenate(pgs, axis=0)
    idx_all = jnp.concatenate(idxs, axis=-1).reshape(_NW, _NB * _B * _R)
    wgt_all = jnp.concatenate(wgts, axis=-1).reshape(_NW, _PT * _R)
    out = _sc_gather(p_total, pg_all, idx_all, wgt_all)
    return out.reshape(n, t, h, w, _KP * _COUT)
